# Initial kernel scaffold; baseline (speedup 1.0000x reference)
import jax
import jax.numpy as jnp
from jax.experimental import pallas as pl


def kernel(x, edge_index, edge_attr, Wl1, Wr1, We1, att1, b1, Wl2, Wr2, We2, att2, b2, Wih, Whh, bih, bhh, Wlin, blin):
    raise NotImplementedError("write your pallas kernel here")


# speedup vs baseline: 6.0852x; 6.0852x over previous
import functools

import jax
import jax.numpy as jnp
from jax import lax
from jax.experimental import pallas as pl
from jax.experimental.pallas import tpu as pltpu
from jax.experimental.pallas import tpu_sc as plsc

T, N, E = 8, 10000, 320000
D_IN, D_EDGE, H, C, NC = 128, 16, 2, 32, 10
HID = H * C

N_TILES = 16
EPT = 20480
EPAD = EPT * N_TILES
CHUNK = 128
N_CHUNKS = EPT // CHUNK
NPAD = 10240
ROWS_PT = NPAD // N_TILES
DUMMY = 10008
ACC_W = 48


def _tables_body(x_ref, wl_ref, wr_ref, xl_ref, xr_ref):
    xb = x_ref[0]
    yl = jnp.dot(xb, wl_ref[...], preferred_element_type=jnp.float32)
    yr = jnp.dot(xb, wr_ref[...], preferred_element_type=jnp.float32)
    xl_ref[0, 0] = yl[:, :C]
    xl_ref[1, 0] = yl[:, C:]
    xr_ref[0, 0] = yr[:, :C]
    xr_ref[1, 0] = yr[:, C:]


def _node_tables(x, wl, wr, nb):
    t, n, d = x.shape
    grid = (t, n // nb)
    out = jax.ShapeDtypeStruct((H, t, n, C), jnp.float32)
    return pl.pallas_call(
        _tables_body,
        grid=grid,
        in_specs=[
            pl.BlockSpec((1, nb, d), lambda i, j: (i, j, 0)),
            pl.BlockSpec((d, HID), lambda i, j: (0, 0)),
            pl.BlockSpec((d, HID), lambda i, j: (0, 0)),
        ],
        out_specs=[
            pl.BlockSpec((H, 1, nb, C), lambda i, j: (0, i, j, 0)),
            pl.BlockSpec((H, 1, nb, C), lambda i, j: (0, i, j, 0)),
        ],
        out_shape=[out, out],
    )(x, wl, wr)


def _ep_body(ea_ref, we1_ref, we2_ref, ep1_ref, ep2_ref):
    eb = ea_ref[0]
    y1 = jnp.dot(eb, we1_ref[...], preferred_element_type=jnp.float32)
    y2 = jnp.dot(eb, we2_ref[...], preferred_element_type=jnp.float32)
    ep1_ref[0, 0] = y1[:, :C]
    ep1_ref[1, 0] = y1[:, C:]
    ep2_ref[0, 0] = y2[:, :C]
    ep2_ref[1, 0] = y2[:, C:]


def _edge_tables(ea, we1, we2, eb):
    t, e, d = ea.shape
    grid = (t, e // eb)
    out = jax.ShapeDtypeStruct((H, t, e, C), jnp.float32)
    return pl.pallas_call(
        _ep_body,
        grid=grid,
        in_specs=[
            pl.BlockSpec((1, eb, d), lambda i, j: (i, j, 0)),
            pl.BlockSpec((d, HID), lambda i, j: (0, 0)),
            pl.BlockSpec((d, HID), lambda i, j: (0, 0)),
        ],
        out_specs=[
            pl.BlockSpec((H, 1, eb, C), lambda i, j: (0, i, j, 0)),
            pl.BlockSpec((H, 1, eb, C), lambda i, j: (0, i, j, 0)),
        ],
        out_shape=[out, out],
    )(ea, we1, we2)


def _elu(v):
    return jnp.where(v > 0, v, jnp.exp(jnp.minimum(v, 0.0)) - 1.0)


def _mid_body(acc_ref, b_ref, wl_ref, wr_ref, xl_ref, xr_ref):
    a0 = acc_ref[0, 0]
    a1 = acc_ref[1, 0]
    h0 = a0[:, :C] / (a0[:, C:C + 1] + 1e-16)
    h1 = a1[:, :C] / (a1[:, C:C + 1] + 1e-16)
    hcat = jnp.concatenate([h0, h1], axis=-1) + b_ref[...]
    hcat = _elu(hcat)
    yl = jnp.dot(hcat, wl_ref[...], preferred_element_type=jnp.float32)
    yr = jnp.dot(hcat, wr_ref[...], preferred_element_type=jnp.float32)
    xl_ref[0, 0] = yl[:, :C]
    xl_ref[1, 0] = yl[:, C:]
    xr_ref[0, 0] = yr[:, :C]
    xr_ref[1, 0] = yr[:, C:]


def _mid_tables(acc, b, wl, wr, nb):
    grid = (T, NPAD // nb)
    out = jax.ShapeDtypeStruct((H, T, NPAD, C), jnp.float32)
    return pl.pallas_call(
        _mid_body,
        grid=grid,
        in_specs=[
            pl.BlockSpec((H, 1, nb, ACC_W), lambda i, j: (0, i, j, 0)),
            pl.BlockSpec((1, HID), lambda i, j: (0, 0)),
            pl.BlockSpec((HID, HID), lambda i, j: (0, 0)),
            pl.BlockSpec((HID, HID), lambda i, j: (0, 0)),
        ],
        out_specs=[
            pl.BlockSpec((H, 1, nb, C), lambda i, j: (0, i, j, 0)),
            pl.BlockSpec((H, 1, nb, C), lambda i, j: (0, i, j, 0)),
        ],
        out_shape=[out, out],
    )(acc, b, wl, wr)


def _embed_body(acc_ref, b_ref, out_ref):
    j = pl.program_id(1)
    a0 = acc_ref[0, 0]
    a1 = acc_ref[1, 0]
    h0 = a0[:, :C] / (a0[:, C:C + 1] + 1e-16)
    h1 = a1[:, :C] / (a1[:, C:C + 1] + 1e-16)
    hcat = _elu(jnp.concatenate([h0, h1], axis=-1) + b_ref[...])
    part = jnp.sum(hcat, axis=0, keepdims=True)[None] * (1.0 / N)

    @pl.when(j == 0)
    def _():
        out_ref[...] = part

    @pl.when(j > 0)
    def _():
        out_ref[...] = out_ref[...] + part


def _embed(acc, b, nb):
    grid = (T, N // nb)
    return pl.pallas_call(
        _embed_body,
        grid=grid,
        in_specs=[
            pl.BlockSpec((H, 1, nb, ACC_W), lambda i, j: (0, i, j, 0)),
            pl.BlockSpec((1, HID), lambda i, j: (0, 0)),
        ],
        out_specs=pl.BlockSpec((1, 1, HID), lambda i, j: (i, 0, 0)),
        out_shape=jax.ShapeDtypeStruct((T, 1, HID), jnp.float32),
    )(acc, b)


def _lstm_body(emb_ref, wih_ref, whh_ref, b_ref, wlin_ref, blin_ref, out_ref):
    zero = jnp.zeros((1, 64), jnp.float32)

    def step(t, carry):
        hh, cc = carry
        xt = emb_ref[pl.ds(t, 1), :]
        g = (jnp.dot(xt, wih_ref[...], preferred_element_type=jnp.float32)
             + jnp.dot(hh, whh_ref[...], preferred_element_type=jnp.float32)
             + b_ref[...])
        i = jax.nn.sigmoid(g[:, 0:64])
        f = jax.nn.sigmoid(g[:, 64:128])
        gg = jnp.tanh(g[:, 128:192])
        o = jax.nn.sigmoid(g[:, 192:256])
        cc = f * cc + i * gg
        hh = o * jnp.tanh(cc)
        return (hh, cc)

    hh, _ = lax.fori_loop(0, T, step, (zero, zero))
    out_ref[...] = jnp.dot(hh, wlin_ref[...],
                           preferred_element_type=jnp.float32) + blin_ref[...]


def _lstm_head(emb, wih_t, whh_t, b, wlin_t, blin):
    return pl.pallas_call(
        _lstm_body,
        out_shape=jax.ShapeDtypeStruct((1, NC), jnp.float32),
    )(emb, wih_t, whh_t, b, wlin_t, blin)


def _sc_edge_pass(xl, xr, ep, src, dst, att):
    mesh = plsc.VectorSubcoreMesh(core_axis_name="c", subcore_axis_name="s")

    @functools.partial(
        pl.kernel,
        out_type=jax.ShapeDtypeStruct((H, T, NPAD, ACC_W), jnp.float32),
        mesh=mesh,
        compiler_params=pltpu.CompilerParams(needs_layout_passes=False, use_tc_tiling_on_sc=False),
        scratch_types=[
            pltpu.VMEM_SHARED((NPAD, C), jnp.float32),
            pltpu.VMEM_SHARED((NPAD, C), jnp.float32),
            pltpu.VMEM_SHARED((NPAD, ACC_W), jnp.float32),
            pltpu.VMEM((CHUNK,), jnp.int32),
            pltpu.VMEM((CHUNK,), jnp.int32),
            pltpu.VMEM((CHUNK, C), jnp.float32),
            pltpu.VMEM((CHUNK, C), jnp.float32),
            pltpu.VMEM((CHUNK, C), jnp.float32),
            pltpu.VMEM((CHUNK, ACC_W), jnp.float32),
            pltpu.VMEM((1, C), jnp.float32),
            pltpu.VMEM((128, ACC_W), jnp.float32),
        ],
    )
    def body(xl_h, xr_h, ep_h, src_h, dst_h, att_h, out_h,
             xl_s, xr_s, acc_s, src_v, dst_v, ep_v, xj_v, xi_v, out_v,
             att_v, zrow_v):
        h = lax.axis_index("c")
        s = lax.axis_index("s")
        row0 = pl.multiple_of(s * ROWS_PT, ROWS_PT)
        iota16 = lax.iota(jnp.int32, 16)
        zeros16 = jnp.zeros((16,), jnp.float32)
        zeros16i = jnp.zeros((16,), jnp.int32)

        pltpu.sync_copy(att_h.at[pl.ds(pl.multiple_of(h * 8, 8), 1)], att_v)

        def zrow_body(r, _):
            zrow_v[r, pl.ds(0, 16)] = zeros16
            zrow_v[r, pl.ds(16, 16)] = zeros16
            zrow_v[r, pl.ds(32, 16)] = zeros16
            return 0

        lax.fori_loop(0, 128, zrow_body, 0)

        def ovpad_body(r, _):
            out_v[r, pl.ds(32, 16)] = zeros16
            return 0

        lax.fori_loop(0, CHUNK, ovpad_body, 0)

        def t_body(t, _):
            pltpu.sync_copy(xl_h.at[h, t, pl.ds(row0, ROWS_PT)],
                            xl_s.at[pl.ds(row0, ROWS_PT)])
            pltpu.sync_copy(xr_h.at[h, t, pl.ds(row0, ROWS_PT)],
                            xr_s.at[pl.ds(row0, ROWS_PT)])
            for z in range(ROWS_PT // 128):
                pltpu.sync_copy(zrow_v, acc_s.at[pl.ds(row0 + 128 * z, 128)])
            plsc.subcore_barrier()

            def chunk_body(k, _):
                base = pl.multiple_of(s * EPT + k * CHUNK, CHUNK)
                pltpu.sync_copy(src_h.at[pl.ds(base, CHUNK)], src_v)
                pltpu.sync_copy(dst_h.at[pl.ds(base, CHUNK)], dst_v)
                pltpu.sync_copy(ep_h.at[h, t, pl.ds(base, CHUNK)], ep_v)
                pltpu.sync_copy(xl_s.at[src_v], xj_v)
                pltpu.sync_copy(xr_s.at[dst_v], xi_v)

                def grp_body(g, _):
                    rows = g * 16 + iota16

                    def ch_body(c, logit):
                        cc = jnp.full((16,), c, jnp.int32)
                        xjv = plsc.load_gather(xj_v, [rows, cc])
                        xiv = plsc.load_gather(xi_v, [rows, cc])
                        epv = plsc.load_gather(ep_v, [rows, cc])
                        av = plsc.load_gather(att_v, [zeros16i, cc])
                        sv = xjv + xiv + epv
                        lr = jnp.maximum(sv, 0.2 * sv)
                        return logit + lr * av

                    logit = lax.fori_loop(0, C, ch_body,
                                          jnp.zeros((16,), jnp.float32))
                    ex = jnp.exp(logit)

                    def sc_body(c, _):
                        cc = jnp.full((16,), c, jnp.int32)
                        xjv = plsc.load_gather(xj_v, [rows, cc])
                        plsc.store_scatter(out_v, [rows, cc], ex * xjv)
                        return 0

                    lax.fori_loop(0, C, sc_body, 0)
                    plsc.store_scatter(out_v,
                                       [rows, jnp.full((16,), C, jnp.int32)],
                                       ex)
                    return 0

                lax.fori_loop(0, CHUNK // 16, grp_body, 0)
                pltpu.sync_copy(out_v, acc_s.at[dst_v], add=True)
                return 0

            lax.fori_loop(0, N_CHUNKS, chunk_body, 0)
            plsc.subcore_barrier()
            pltpu.sync_copy(acc_s.at[pl.ds(row0, ROWS_PT)],
                            out_h.at[h, t, pl.ds(row0, ROWS_PT)])
            return 0

        lax.fori_loop(0, T, t_body, 0)

    return body(xl, xr, ep, src, dst, att)


def kernel(x, edge_index, edge_attr, Wl1, Wr1, We1, att1, b1,
           Wl2, Wr2, We2, att2, b2, Wih, Whh, bih, bhh, Wlin, blin):
    src = jnp.pad(edge_index[0], (0, EPAD - E))
    dst = jnp.pad(edge_index[1], (0, EPAD - E), constant_values=DUMMY)
    ea = jnp.pad(edge_attr, ((0, 0), (0, EPAD - E), (0, 0)))
    xp = jnp.pad(x, ((0, 0), (0, NPAD - N), (0, 0)))
    att1p = jnp.zeros((16, C), jnp.float32).at[0].set(att1[0]).at[8].set(att1[1])
    att2p = jnp.zeros((16, C), jnp.float32).at[0].set(att2[0]).at[8].set(att2[1])

    xl1, xr1 = _node_tables(xp, Wl1, Wr1, nb=2048)
    ep1, ep2 = _edge_tables(ea, We1, We2, eb=4096)

    acc1 = _sc_edge_pass(xl1, xr1, ep1, src, dst, att1p)
    xl2, xr2 = _mid_tables(acc1, b1.reshape(1, HID), Wl2, Wr2, nb=2048)
    acc2 = _sc_edge_pass(xl2, xr2, ep2, src, dst, att2p)

    emb = _embed(acc2, b2.reshape(1, HID), nb=2000).reshape(T, HID)
    out = _lstm_head(emb, Wih.T, Whh.T, (bih + bhh).reshape(1, 4 * 64),
                     Wlin.T, blin.reshape(1, NC))
    return out.reshape(NC)


# speedup vs baseline: 7.3444x; 1.2069x over previous
import functools

import jax
import jax.numpy as jnp
from jax import lax
from jax.experimental import pallas as pl
from jax.experimental.pallas import tpu as pltpu
from jax.experimental.pallas import tpu_sc as plsc

T, N, E = 8, 10000, 320000
D_IN, D_EDGE, H, C, NC = 128, 16, 2, 32, 10
HID = H * C

N_TILES = 16
EPT = 20480
EPAD = EPT * N_TILES
CHUNK = 128
N_CHUNKS = EPT // CHUNK
NPAD = 10240
ROWS_PT = NPAD // N_TILES
DUMMY = 10008
ACC_W = 48


def _tables_body(x_ref, wl_ref, wr_ref, xl_ref, xr_ref):
    xb = x_ref[0]
    yl = jnp.dot(xb, wl_ref[...], preferred_element_type=jnp.float32)
    yr = jnp.dot(xb, wr_ref[...], preferred_element_type=jnp.float32)
    xl_ref[0, 0] = yl[:, :C]
    xl_ref[1, 0] = yl[:, C:]
    xr_ref[0, 0] = yr[:, :C]
    xr_ref[1, 0] = yr[:, C:]


def _node_tables(x, wl, wr, nb):
    t, n, d = x.shape
    grid = (t, n // nb)
    out = jax.ShapeDtypeStruct((H, t, n, C), jnp.float32)
    return pl.pallas_call(
        _tables_body,
        grid=grid,
        in_specs=[
            pl.BlockSpec((1, nb, d), lambda i, j: (i, j, 0)),
            pl.BlockSpec((d, HID), lambda i, j: (0, 0)),
            pl.BlockSpec((d, HID), lambda i, j: (0, 0)),
        ],
        out_specs=[
            pl.BlockSpec((H, 1, nb, C), lambda i, j: (0, i, j, 0)),
            pl.BlockSpec((H, 1, nb, C), lambda i, j: (0, i, j, 0)),
        ],
        out_shape=[out, out],
    )(x, wl, wr)


def _ep_body(ea_ref, we1_ref, we2_ref, ep1_ref, ep2_ref):
    eb = ea_ref[0]
    y1 = jnp.dot(eb, we1_ref[...], preferred_element_type=jnp.float32)
    y2 = jnp.dot(eb, we2_ref[...], preferred_element_type=jnp.float32)
    ep1_ref[0, 0] = y1[:, :C]
    ep1_ref[1, 0] = y1[:, C:]
    ep2_ref[0, 0] = y2[:, :C]
    ep2_ref[1, 0] = y2[:, C:]


def _edge_tables(ea, we1, we2, eb):
    t, e, d = ea.shape
    grid = (t, e // eb)
    out = jax.ShapeDtypeStruct((H, t, e, C), jnp.float32)
    return pl.pallas_call(
        _ep_body,
        grid=grid,
        in_specs=[
            pl.BlockSpec((1, eb, d), lambda i, j: (i, j, 0)),
            pl.BlockSpec((d, HID), lambda i, j: (0, 0)),
            pl.BlockSpec((d, HID), lambda i, j: (0, 0)),
        ],
        out_specs=[
            pl.BlockSpec((H, 1, eb, C), lambda i, j: (0, i, j, 0)),
            pl.BlockSpec((H, 1, eb, C), lambda i, j: (0, i, j, 0)),
        ],
        out_shape=[out, out],
    )(ea, we1, we2)


def _elu(v):
    return jnp.where(v > 0, v, jnp.exp(jnp.minimum(v, 0.0)) - 1.0)


def _mid_body(acc_ref, b_ref, wl_ref, wr_ref, xl_ref, xr_ref):
    a0 = acc_ref[0, 0]
    a1 = acc_ref[1, 0]
    h0 = a0[:, :C] / (a0[:, C:C + 1] + 1e-16)
    h1 = a1[:, :C] / (a1[:, C:C + 1] + 1e-16)
    hcat = jnp.concatenate([h0, h1], axis=-1) + b_ref[...]
    hcat = _elu(hcat)
    yl = jnp.dot(hcat, wl_ref[...], preferred_element_type=jnp.float32)
    yr = jnp.dot(hcat, wr_ref[...], preferred_element_type=jnp.float32)
    xl_ref[0, 0] = yl[:, :C]
    xl_ref[1, 0] = yl[:, C:]
    xr_ref[0, 0] = yr[:, :C]
    xr_ref[1, 0] = yr[:, C:]


def _mid_tables(acc, b, wl, wr, nb):
    grid = (T, NPAD // nb)
    out = jax.ShapeDtypeStruct((H, T, NPAD, C), jnp.float32)
    return pl.pallas_call(
        _mid_body,
        grid=grid,
        in_specs=[
            pl.BlockSpec((H, 1, nb, ACC_W), lambda i, j: (0, i, j, 0)),
            pl.BlockSpec((1, HID), lambda i, j: (0, 0)),
            pl.BlockSpec((HID, HID), lambda i, j: (0, 0)),
            pl.BlockSpec((HID, HID), lambda i, j: (0, 0)),
        ],
        out_specs=[
            pl.BlockSpec((H, 1, nb, C), lambda i, j: (0, i, j, 0)),
            pl.BlockSpec((H, 1, nb, C), lambda i, j: (0, i, j, 0)),
        ],
        out_shape=[out, out],
    )(acc, b, wl, wr)


def _embed_body(acc_ref, b_ref, out_ref):
    j = pl.program_id(1)
    a0 = acc_ref[0, 0]
    a1 = acc_ref[1, 0]
    h0 = a0[:, :C] / (a0[:, C:C + 1] + 1e-16)
    h1 = a1[:, :C] / (a1[:, C:C + 1] + 1e-16)
    hcat = _elu(jnp.concatenate([h0, h1], axis=-1) + b_ref[...])
    part = jnp.sum(hcat, axis=0, keepdims=True)[None] * (1.0 / N)

    @pl.when(j == 0)
    def _():
        out_ref[...] = part

    @pl.when(j > 0)
    def _():
        out_ref[...] = out_ref[...] + part


def _embed(acc, b, nb):
    grid = (T, N // nb)
    return pl.pallas_call(
        _embed_body,
        grid=grid,
        in_specs=[
            pl.BlockSpec((H, 1, nb, ACC_W), lambda i, j: (0, i, j, 0)),
            pl.BlockSpec((1, HID), lambda i, j: (0, 0)),
        ],
        out_specs=pl.BlockSpec((1, 1, HID), lambda i, j: (i, 0, 0)),
        out_shape=jax.ShapeDtypeStruct((T, 1, HID), jnp.float32),
    )(acc, b)


def _lstm_body(emb_ref, wih_ref, whh_ref, b_ref, wlin_ref, blin_ref, out_ref):
    zero = jnp.zeros((1, 64), jnp.float32)

    def step(t, carry):
        hh, cc = carry
        xt = emb_ref[pl.ds(t, 1), :]
        g = (jnp.dot(xt, wih_ref[...], preferred_element_type=jnp.float32)
             + jnp.dot(hh, whh_ref[...], preferred_element_type=jnp.float32)
             + b_ref[...])
        i = jax.nn.sigmoid(g[:, 0:64])
        f = jax.nn.sigmoid(g[:, 64:128])
        gg = jnp.tanh(g[:, 128:192])
        o = jax.nn.sigmoid(g[:, 192:256])
        cc = f * cc + i * gg
        hh = o * jnp.tanh(cc)
        return (hh, cc)

    hh, _ = lax.fori_loop(0, T, step, (zero, zero))
    out_ref[...] = jnp.dot(hh, wlin_ref[...],
                           preferred_element_type=jnp.float32) + blin_ref[...]


def _lstm_head(emb, wih_t, whh_t, b, wlin_t, blin):
    return pl.pallas_call(
        _lstm_body,
        out_shape=jax.ShapeDtypeStruct((1, NC), jnp.float32),
    )(emb, wih_t, whh_t, b, wlin_t, blin)


def _sc_edge_pass(xl, xr, ep, src, dst, att):
    mesh = plsc.VectorSubcoreMesh(core_axis_name="c", subcore_axis_name="s")

    @functools.partial(
        pl.kernel,
        out_type=jax.ShapeDtypeStruct((H, T, NPAD, ACC_W), jnp.float32),
        mesh=mesh,
        compiler_params=pltpu.CompilerParams(
            needs_layout_passes=False, use_tc_tiling_on_sc=False),
        scratch_types=[
            pltpu.VMEM_SHARED((NPAD, ACC_W), jnp.float32),
            pltpu.VMEM((N_CHUNKS, CHUNK), jnp.int32),
            pltpu.VMEM((N_CHUNKS, CHUNK), jnp.int32),
            pltpu.VMEM((2, CHUNK, C), jnp.float32),
            pltpu.VMEM((2, CHUNK, C), jnp.float32),
            pltpu.VMEM((2, CHUNK, C), jnp.float32),
            pltpu.VMEM((2, CHUNK, ACC_W), jnp.float32),
            pltpu.VMEM((1, C), jnp.float32),
            pltpu.VMEM((128, ACC_W), jnp.float32),
            pltpu.SemaphoreType.DMA((2,)),
            pltpu.SemaphoreType.DMA((2,)),
            pltpu.SemaphoreType.DMA((2,)),
            pltpu.SemaphoreType.DMA((2,)),
        ],
    )
    def body(xl_h, xr_h, ep_h, src_h, dst_h, att_h, out_h,
             acc_s, src_v, dst_v, ep_v, xj_v, xi_v, out_v,
             att_v, zrow_v, sem_e, sem_j, sem_i, sem_s):
        h = lax.axis_index("c")
        s = lax.axis_index("s")
        row0 = pl.multiple_of(s * ROWS_PT, ROWS_PT)
        iota16 = lax.iota(jnp.int32, 16)
        zeros16 = jnp.zeros((16,), jnp.float32)
        zeros16i = jnp.zeros((16,), jnp.int32)

        pltpu.sync_copy(att_h.at[pl.ds(pl.multiple_of(h * 8, 8), 1)], att_v)
        crow0 = pl.multiple_of(s * N_CHUNKS, N_CHUNKS)
        pltpu.sync_copy(src_h.at[pl.ds(crow0, N_CHUNKS)], src_v)
        pltpu.sync_copy(dst_h.at[pl.ds(crow0, N_CHUNKS)], dst_v)

        def zrow_body(r, _):
            zrow_v[r, pl.ds(0, 16)] = zeros16
            zrow_v[r, pl.ds(16, 16)] = zeros16
            zrow_v[r, pl.ds(32, 16)] = zeros16
            return 0

        lax.fori_loop(0, 128, zrow_body, 0)

        def ovpad_body(r, _):
            out_v[0, r, pl.ds(32, 16)] = zeros16
            out_v[1, r, pl.ds(32, 16)] = zeros16
            return 0

        lax.fori_loop(0, CHUNK, ovpad_body, 0)

        def start_fetch(t, ch, b):
            base = pl.multiple_of(s * EPT, EPT) + ch * CHUNK
            pltpu.async_copy(ep_h.at[h, t, pl.ds(base, CHUNK)],
                             ep_v.at[b], sem_e.at[b])
            pltpu.async_copy(xl_h.at[h, t].at[src_v.at[ch]], xj_v.at[b],
                             sem_j.at[b])
            pltpu.async_copy(xr_h.at[h, t].at[dst_v.at[ch]], xi_v.at[b],
                             sem_i.at[b])

        def wait_fetch(t, b):
            pltpu.make_async_copy(ep_h.at[h, t, pl.ds(0, CHUNK)],
                                  ep_v.at[b], sem_e.at[b]).wait()
            pltpu.make_async_copy(xl_h.at[h, t].at[src_v.at[0]],
                                  xj_v.at[b], sem_j.at[b]).wait()
            pltpu.make_async_copy(xr_h.at[h, t].at[dst_v.at[0]],
                                  xi_v.at[b], sem_i.at[b]).wait()

        def compute_chunk(b):
            xj_b = xj_v.at[b]
            xi_b = xi_v.at[b]
            ep_b = ep_v.at[b]
            out_b = out_v.at[b]

            def grp_body(g, _):
                rows = g * 16 + iota16

                def ch_body(c, logit):
                    cc = jnp.full((16,), c, jnp.int32)
                    xjv = plsc.load_gather(xj_b, [rows, cc])
                    xiv = plsc.load_gather(xi_b, [rows, cc])
                    epv = plsc.load_gather(ep_b, [rows, cc])
                    av = plsc.load_gather(att_v, [zeros16i, cc])
                    sv = xjv + xiv + epv
                    lr = jnp.maximum(sv, 0.2 * sv)
                    return logit + lr * av

                logit = lax.fori_loop(0, C, ch_body,
                                      jnp.zeros((16,), jnp.float32))
                ex = jnp.exp(logit)

                def sc_body(c, _):
                    cc = jnp.full((16,), c, jnp.int32)
                    xjv = plsc.load_gather(xj_b, [rows, cc])
                    plsc.store_scatter(out_b, [rows, cc], ex * xjv)
                    return 0

                lax.fori_loop(0, C, sc_body, 0)
                plsc.store_scatter(out_b,
                                   [rows, jnp.full((16,), C, jnp.int32)],
                                   ex)
                return 0

            lax.fori_loop(0, CHUNK // 16, grp_body, 0)

        def t_body(t, _):
            for z in range(ROWS_PT // 128):
                pltpu.sync_copy(zrow_v, acc_s.at[pl.ds(row0 + 128 * z, 128)])
            plsc.subcore_barrier()

            start_fetch(t, 0, 0)
            start_fetch(t, 1, 1)
            last = N_CHUNKS - 1

            def pair_body(k2, _):
                a = 2 * k2
                for b in range(2):
                    ch = a + b
                    wait_fetch(t, b)

                    @pl.when(k2 > 0)
                    def _():
                        pltpu.make_async_copy(
                            out_v.at[b], acc_s.at[dst_v.at[0]],
                            sem_s.at[b]).wait()

                    compute_chunk(b)
                    pltpu.async_copy(out_v.at[b], acc_s.at[dst_v.at[ch]],
                                     sem_s.at[b], add=True)
                    nxt = jnp.minimum(ch + 2, last)
                    start_fetch(t, nxt, b)
                return 0

            lax.fori_loop(0, N_CHUNKS // 2, pair_body, 0)
            for b in range(2):
                pltpu.make_async_copy(out_v.at[b], acc_s.at[dst_v.at[0]],
                                      sem_s.at[b]).wait()
                wait_fetch(t, b)
            plsc.subcore_barrier()
            pltpu.sync_copy(acc_s.at[pl.ds(row0, ROWS_PT)],
                            out_h.at[h, t, pl.ds(row0, ROWS_PT)])
            return 0

        lax.fori_loop(0, T, t_body, 0)

    return body(xl, xr, ep, src, dst, att)


def kernel(x, edge_index, edge_attr, Wl1, Wr1, We1, att1, b1,
           Wl2, Wr2, We2, att2, b2, Wih, Whh, bih, bhh, Wlin, blin):
    src = jnp.pad(edge_index[0], (0, EPAD - E)).reshape(
        N_TILES * N_CHUNKS, CHUNK)
    dst = jnp.pad(edge_index[1], (0, EPAD - E),
                  constant_values=DUMMY).reshape(N_TILES * N_CHUNKS, CHUNK)
    ea = jnp.pad(edge_attr, ((0, 0), (0, EPAD - E), (0, 0)))
    xp = jnp.pad(x, ((0, 0), (0, NPAD - N), (0, 0)))
    att1p = jnp.zeros((16, C), jnp.float32).at[0].set(att1[0]).at[8].set(att1[1])
    att2p = jnp.zeros((16, C), jnp.float32).at[0].set(att2[0]).at[8].set(att2[1])

    xl1, xr1 = _node_tables(xp, Wl1, Wr1, nb=2048)
    ep1, ep2 = _edge_tables(ea, We1, We2, eb=4096)

    acc1 = _sc_edge_pass(xl1, xr1, ep1, src, dst, att1p)
    xl2, xr2 = _mid_tables(acc1, b1.reshape(1, HID), Wl2, Wr2, nb=2048)
    acc2 = _sc_edge_pass(xl2, xr2, ep2, src, dst, att2p)

    emb = _embed(acc2, b2.reshape(1, HID), nb=2000).reshape(T, HID)
    out = _lstm_head(emb, Wih.T, Whh.T, (bih + bhh).reshape(1, 4 * 64),
                     Wlin.T, blin.reshape(1, NC))
    return out.reshape(NC)


# speedup vs baseline: 19.9834x; 2.7209x over previous
import functools

import jax
import jax.numpy as jnp
from jax import lax
from jax.experimental import pallas as pl
from jax.experimental.pallas import tpu as pltpu
from jax.experimental.pallas import tpu_sc as plsc

T, N, E = 8, 10000, 320000
D_IN, D_EDGE, H, C, NC = 128, 16, 2, 32, 10
HID = H * C

N_TILES = 16
EPT = 20480
EPAD = EPT * N_TILES
CHUNK = 128
N_CHUNKS = EPT // CHUNK
NPAD = 10240
ROWS_PT = NPAD // N_TILES
DUMMY = 10008
ACC_W = 48


def _tables_body(x_ref, wl_ref, wr_ref, xl_ref, xr_ref):
    xb = x_ref[0]
    yl = jnp.dot(xb, wl_ref[...], preferred_element_type=jnp.float32)
    yr = jnp.dot(xb, wr_ref[...], preferred_element_type=jnp.float32)
    xl_ref[0, 0] = yl[:, :C]
    xl_ref[1, 0] = yl[:, C:]
    xr_ref[0, 0] = yr[:, :C]
    xr_ref[1, 0] = yr[:, C:]


def _node_tables(x, wl, wr, nb):
    t, n, d = x.shape
    grid = (t, n // nb)
    out = jax.ShapeDtypeStruct((H, t, n, C), jnp.float32)
    return pl.pallas_call(
        _tables_body,
        grid=grid,
        in_specs=[
            pl.BlockSpec((1, nb, d), lambda i, j: (i, j, 0)),
            pl.BlockSpec((d, HID), lambda i, j: (0, 0)),
            pl.BlockSpec((d, HID), lambda i, j: (0, 0)),
        ],
        out_specs=[
            pl.BlockSpec((H, 1, nb, C), lambda i, j: (0, i, j, 0)),
            pl.BlockSpec((H, 1, nb, C), lambda i, j: (0, i, j, 0)),
        ],
        out_shape=[out, out],
    )(x, wl, wr)


def _ep_body(ea_ref, we1_ref, we2_ref, ep1_ref, ep2_ref):
    eb = ea_ref[0]
    y1 = jnp.dot(eb, we1_ref[...], preferred_element_type=jnp.float32)
    y2 = jnp.dot(eb, we2_ref[...], preferred_element_type=jnp.float32)
    ep1_ref[0, 0] = y1[:, :C]
    ep1_ref[1, 0] = y1[:, C:]
    ep2_ref[0, 0] = y2[:, :C]
    ep2_ref[1, 0] = y2[:, C:]


def _edge_tables(ea, we1, we2, eb):
    t, e, d = ea.shape
    grid = (t, e // eb)
    out = jax.ShapeDtypeStruct((H, t, e, C), jnp.float32)
    return pl.pallas_call(
        _ep_body,
        grid=grid,
        in_specs=[
            pl.BlockSpec((1, eb, d), lambda i, j: (i, j, 0)),
            pl.BlockSpec((d, HID), lambda i, j: (0, 0)),
            pl.BlockSpec((d, HID), lambda i, j: (0, 0)),
        ],
        out_specs=[
            pl.BlockSpec((H, 1, eb, C), lambda i, j: (0, i, j, 0)),
            pl.BlockSpec((H, 1, eb, C), lambda i, j: (0, i, j, 0)),
        ],
        out_shape=[out, out],
    )(ea, we1, we2)


def _elu(v):
    return jnp.where(v > 0, v, jnp.exp(jnp.minimum(v, 0.0)) - 1.0)


def _mid_body(acc_ref, b_ref, wl_ref, wr_ref, xl_ref, xr_ref):
    a0 = acc_ref[0, 0]
    a1 = acc_ref[1, 0]
    h0 = a0[:, :C] / (a0[:, C:C + 1] + 1e-16)
    h1 = a1[:, :C] / (a1[:, C:C + 1] + 1e-16)
    hcat = jnp.concatenate([h0, h1], axis=-1) + b_ref[...]
    hcat = _elu(hcat)
    yl = jnp.dot(hcat, wl_ref[...], preferred_element_type=jnp.float32)
    yr = jnp.dot(hcat, wr_ref[...], preferred_element_type=jnp.float32)
    xl_ref[0, 0] = yl[:, :C]
    xl_ref[1, 0] = yl[:, C:]
    xr_ref[0, 0] = yr[:, :C]
    xr_ref[1, 0] = yr[:, C:]


def _mid_tables(acc, b, wl, wr, nb):
    grid = (T, NPAD // nb)
    out = jax.ShapeDtypeStruct((H, T, NPAD, C), jnp.float32)
    return pl.pallas_call(
        _mid_body,
        grid=grid,
        in_specs=[
            pl.BlockSpec((H, 1, nb, ACC_W), lambda i, j: (0, i, j, 0)),
            pl.BlockSpec((1, HID), lambda i, j: (0, 0)),
            pl.BlockSpec((HID, HID), lambda i, j: (0, 0)),
            pl.BlockSpec((HID, HID), lambda i, j: (0, 0)),
        ],
        out_specs=[
            pl.BlockSpec((H, 1, nb, C), lambda i, j: (0, i, j, 0)),
            pl.BlockSpec((H, 1, nb, C), lambda i, j: (0, i, j, 0)),
        ],
        out_shape=[out, out],
    )(acc, b, wl, wr)


def _embed_body(acc_ref, b_ref, out_ref):
    j = pl.program_id(1)
    a0 = acc_ref[0, 0]
    a1 = acc_ref[1, 0]
    h0 = a0[:, :C] / (a0[:, C:C + 1] + 1e-16)
    h1 = a1[:, :C] / (a1[:, C:C + 1] + 1e-16)
    hcat = _elu(jnp.concatenate([h0, h1], axis=-1) + b_ref[...])
    part = jnp.sum(hcat, axis=0, keepdims=True)[None] * (1.0 / N)

    @pl.when(j == 0)
    def _():
        out_ref[...] = part

    @pl.when(j > 0)
    def _():
        out_ref[...] = out_ref[...] + part


def _embed(acc, b, nb):
    grid = (T, N // nb)
    return pl.pallas_call(
        _embed_body,
        grid=grid,
        in_specs=[
            pl.BlockSpec((H, 1, nb, ACC_W), lambda i, j: (0, i, j, 0)),
            pl.BlockSpec((1, HID), lambda i, j: (0, 0)),
        ],
        out_specs=pl.BlockSpec((1, 1, HID), lambda i, j: (i, 0, 0)),
        out_shape=jax.ShapeDtypeStruct((T, 1, HID), jnp.float32),
    )(acc, b)


def _lstm_body(emb_ref, wih_ref, whh_ref, b_ref, wlin_ref, blin_ref, out_ref):
    zero = jnp.zeros((1, 64), jnp.float32)

    def step(t, carry):
        hh, cc = carry
        xt = emb_ref[pl.ds(t, 1), :]
        g = (jnp.dot(xt, wih_ref[...], preferred_element_type=jnp.float32)
             + jnp.dot(hh, whh_ref[...], preferred_element_type=jnp.float32)
             + b_ref[...])
        i = jax.nn.sigmoid(g[:, 0:64])
        f = jax.nn.sigmoid(g[:, 64:128])
        gg = jnp.tanh(g[:, 128:192])
        o = jax.nn.sigmoid(g[:, 192:256])
        cc = f * cc + i * gg
        hh = o * jnp.tanh(cc)
        return (hh, cc)

    hh, _ = lax.fori_loop(0, T, step, (zero, zero))
    out_ref[...] = jnp.dot(hh, wlin_ref[...],
                           preferred_element_type=jnp.float32) + blin_ref[...]


def _lstm_head(emb, wih_t, whh_t, b, wlin_t, blin):
    return pl.pallas_call(
        _lstm_body,
        out_shape=jax.ShapeDtypeStruct((1, NC), jnp.float32),
    )(emb, wih_t, whh_t, b, wlin_t, blin)


def _sc_edge_pass(xl, xr, ep, src, dst, att):
    mesh = plsc.VectorSubcoreMesh(core_axis_name="c", subcore_axis_name="s")

    @functools.partial(
        pl.kernel,
        out_type=jax.ShapeDtypeStruct((H, T, NPAD, ACC_W), jnp.float32),
        mesh=mesh,
        compiler_params=pltpu.CompilerParams(
            needs_layout_passes=False, use_tc_tiling_on_sc=False),
        scratch_types=[
            pltpu.VMEM_SHARED((NPAD, ACC_W), jnp.float32),
            pltpu.VMEM((N_CHUNKS, CHUNK), jnp.int32),
            pltpu.VMEM((N_CHUNKS, CHUNK), jnp.int32),
            pltpu.VMEM((2, CHUNK, C), jnp.float32),
            pltpu.VMEM((2, CHUNK, C), jnp.float32),
            pltpu.VMEM((2, CHUNK, C), jnp.float32),
            pltpu.VMEM((2, CHUNK, ACC_W), jnp.float32),
            pltpu.VMEM((1, C), jnp.float32),
            pltpu.VMEM((128, ACC_W), jnp.float32),
            pltpu.SemaphoreType.DMA((2,)),
            pltpu.SemaphoreType.DMA((2,)),
            pltpu.SemaphoreType.DMA((2,)),
            pltpu.SemaphoreType.DMA((2,)),
        ],
    )
    def body(xl_h, xr_h, ep_h, src_h, dst_h, att_h, out_h,
             acc_s, src_v, dst_v, ep_v, xj_v, xi_v, out_v,
             att_v, zrow_v, sem_e, sem_j, sem_i, sem_s):
        h = lax.axis_index("c")
        s = lax.axis_index("s")
        row0 = pl.multiple_of(s * ROWS_PT, ROWS_PT)
        iota16 = lax.iota(jnp.int32, 16)
        zeros16 = jnp.zeros((16,), jnp.float32)
        zeros16i = jnp.zeros((16,), jnp.int32)

        pltpu.sync_copy(att_h.at[pl.ds(pl.multiple_of(h * 8, 8), 1)], att_v)
        crow0 = pl.multiple_of(s * N_CHUNKS, N_CHUNKS)
        pltpu.sync_copy(src_h.at[pl.ds(crow0, N_CHUNKS)], src_v)
        pltpu.sync_copy(dst_h.at[pl.ds(crow0, N_CHUNKS)], dst_v)

        def zrow_body(r, _):
            zrow_v[r, pl.ds(0, 16)] = zeros16
            zrow_v[r, pl.ds(16, 16)] = zeros16
            zrow_v[r, pl.ds(32, 16)] = zeros16
            return 0

        lax.fori_loop(0, 128, zrow_body, 0)

        def ovpad_body(r, _):
            out_v[0, r, pl.ds(32, 16)] = zeros16
            out_v[1, r, pl.ds(32, 16)] = zeros16
            return 0

        lax.fori_loop(0, CHUNK, ovpad_body, 0)

        def start_fetch(t, ch, b):
            base = pl.multiple_of(s * EPT, EPT) + ch * CHUNK
            pltpu.async_copy(ep_h.at[h, t, pl.ds(base, CHUNK)],
                             ep_v.at[b], sem_e.at[b])
            pltpu.async_copy(xl_h.at[h, t].at[src_v.at[ch]], xj_v.at[b],
                             sem_j.at[b])
            pltpu.async_copy(xr_h.at[h, t].at[dst_v.at[ch]], xi_v.at[b],
                             sem_i.at[b])

        def wait_fetch(t, b):
            pltpu.make_async_copy(ep_h.at[h, t, pl.ds(0, CHUNK)],
                                  ep_v.at[b], sem_e.at[b]).wait()
            pltpu.make_async_copy(xl_h.at[h, t].at[src_v.at[0]],
                                  xj_v.at[b], sem_j.at[b]).wait()
            pltpu.make_async_copy(xr_h.at[h, t].at[dst_v.at[0]],
                                  xi_v.at[b], sem_i.at[b]).wait()

        def compute_chunk(b):
            xj_b = xj_v.at[b]
            xi_b = xi_v.at[b]
            ep_b = ep_v.at[b]
            out_b = out_v.at[b]
            def grp_body(g, _):
                rows = g * 16 + iota16

                def ch4_body(c4, logit):
                    c0 = c4 * 4
                    for u in range(4):
                        cd = (c0 + u + iota16) & (C - 1)
                        xjv = plsc.load_gather(xj_b, [rows, cd])
                        xiv = plsc.load_gather(xi_b, [rows, cd])
                        epv = plsc.load_gather(ep_b, [rows, cd])
                        av = plsc.load_gather(att_v, [zeros16i, cd])
                        sv = xjv + xiv + epv
                        lr = jnp.maximum(sv, 0.2 * sv)
                        logit = logit + lr * av
                    return logit

                logit = lax.fori_loop(0, C // 4, ch4_body,
                                      jnp.zeros((16,), jnp.float32))
                ex = jnp.exp(logit)

                def sc4_body(c4, _):
                    c0 = c4 * 4
                    for u in range(4):
                        cd = (c0 + u + iota16) & (C - 1)
                        xjv = plsc.load_gather(xj_b, [rows, cd])
                        plsc.store_scatter(out_b, [rows, cd], ex * xjv)
                    return 0

                lax.fori_loop(0, C // 4, sc4_body, 0)
                plsc.store_scatter(out_b,
                                   [rows, jnp.full((16,), C, jnp.int32)],
                                   ex)
                return 0

            lax.fori_loop(0, CHUNK // 16, grp_body, 0)

        def t_body(t, _):
            for z in range(ROWS_PT // 128):
                pltpu.sync_copy(zrow_v, acc_s.at[pl.ds(row0 + 128 * z, 128)])
            plsc.subcore_barrier()

            start_fetch(t, 0, 0)
            start_fetch(t, 1, 1)
            last = N_CHUNKS - 1

            def pair_body(k2, _):
                a = 2 * k2
                for b in range(2):
                    ch = a + b
                    wait_fetch(t, b)

                    @pl.when(k2 > 0)
                    def _():
                        pltpu.make_async_copy(
                            out_v.at[b], acc_s.at[dst_v.at[0]],
                            sem_s.at[b]).wait()

                    compute_chunk(b)
                    pltpu.async_copy(out_v.at[b], acc_s.at[dst_v.at[ch]],
                                     sem_s.at[b], add=True)
                    nxt = jnp.minimum(ch + 2, last)
                    start_fetch(t, nxt, b)
                return 0

            lax.fori_loop(0, N_CHUNKS // 2, pair_body, 0)
            for b in range(2):
                pltpu.make_async_copy(out_v.at[b], acc_s.at[dst_v.at[0]],
                                      sem_s.at[b]).wait()
                wait_fetch(t, b)
            plsc.subcore_barrier()
            pltpu.sync_copy(acc_s.at[pl.ds(row0, ROWS_PT)],
                            out_h.at[h, t, pl.ds(row0, ROWS_PT)])
            return 0

        lax.fori_loop(0, T, t_body, 0)

    return body(xl, xr, ep, src, dst, att)


def kernel(x, edge_index, edge_attr, Wl1, Wr1, We1, att1, b1,
           Wl2, Wr2, We2, att2, b2, Wih, Whh, bih, bhh, Wlin, blin):
    src = jnp.pad(edge_index[0], (0, EPAD - E)).reshape(
        N_TILES * N_CHUNKS, CHUNK)
    dst = jnp.pad(edge_index[1], (0, EPAD - E),
                  constant_values=DUMMY).reshape(N_TILES * N_CHUNKS, CHUNK)
    ea = jnp.pad(edge_attr, ((0, 0), (0, EPAD - E), (0, 0)))
    xp = jnp.pad(x, ((0, 0), (0, NPAD - N), (0, 0)))
    att1p = jnp.zeros((16, C), jnp.float32).at[0].set(att1[0]).at[8].set(att1[1])
    att2p = jnp.zeros((16, C), jnp.float32).at[0].set(att2[0]).at[8].set(att2[1])

    xl1, xr1 = _node_tables(xp, Wl1, Wr1, nb=2048)
    ep1, ep2 = _edge_tables(ea, We1, We2, eb=4096)

    acc1 = _sc_edge_pass(xl1, xr1, ep1, src, dst, att1p)
    xl2, xr2 = _mid_tables(acc1, b1.reshape(1, HID), Wl2, Wr2, nb=2048)
    acc2 = _sc_edge_pass(xl2, xr2, ep2, src, dst, att2p)

    emb = _embed(acc2, b2.reshape(1, HID), nb=2000).reshape(T, HID)
    out = _lstm_head(emb, Wih.T, Whh.T, (bih + bhh).reshape(1, 4 * 64),
                     Wlin.T, blin.reshape(1, NC))
    return out.reshape(NC)
